# Initial kernel scaffold; baseline (speedup 1.0000x reference)
import jax
import jax.numpy as jnp
from jax.experimental import pallas as pl


def kernel(x, edge_index, edge_weight, W1, b1, W2, b2, a, b, c, d):
    raise NotImplementedError("write your pallas kernel here")


# speedup vs baseline: 2.8576x; 2.8576x over previous
import functools

import jax
import jax.numpy as jnp
from jax import lax
from jax.experimental import pallas as pl
from jax.experimental.pallas import tpu as pltpu
from jax.experimental.pallas import tpu_sc as plsc

N_NODES = 10000
D_IN = 128
HID = 256
D_OUT = 128
N_SUBCORES = 16
CHUNK = 128


_BLK_M = 2000


def _mm1_body(x_ref, w_ref, o_ref):
    o_ref[0] = jnp.dot(x_ref[...], w_ref[...], preferred_element_type=jnp.float32)


def _matmul_split(x, w):
    n, k = x.shape
    fh = w.shape[1] // 2
    grid = (n // _BLK_M, 2)
    return pl.pallas_call(
        _mm1_body,
        grid=grid,
        in_specs=[
            pl.BlockSpec((_BLK_M, k), lambda i, c: (i, 0)),
            pl.BlockSpec((k, fh), lambda i, c: (0, c)),
        ],
        out_specs=pl.BlockSpec((1, _BLK_M, fh), lambda i, c: (c, i, 0)),
        out_shape=jax.ShapeDtypeStruct((2, n, fh), jnp.float32),
    )(x, w)


def _grelu(x, ga, gb, gc, gd):
    out = jnp.where(x < 0, ga * x, x)
    out = jnp.where((x >= 0) & (x < gc), gb * x, out)
    out = jnp.where(x >= gc, gd * x, out)
    return out


def _mm2_body(p_ref, agg_ref, w_ref, o_ref):
    ga, gb, gc, gd = p_ref[0], p_ref[1], p_ref[2], p_ref[3]
    a0 = _grelu(agg_ref[0], ga, gb, gc, gd)
    a1 = _grelu(agg_ref[1], ga, gb, gc, gd)
    k = a0.shape[1]
    o_ref[...] = jnp.dot(a0, w_ref[:k, :], preferred_element_type=jnp.float32) + jnp.dot(
        a1, w_ref[k:, :], preferred_element_type=jnp.float32
    )


def _matmul2_full(params, agg, w):
    _, n, kh = agg.shape
    f = w.shape[1]
    grid = (n // _BLK_M,)
    return pl.pallas_call(
        _mm2_body,
        grid=grid,
        in_specs=[
            pl.BlockSpec(memory_space=pltpu.SMEM),
            pl.BlockSpec((2, _BLK_M, kh), lambda i: (0, i, 0)),
            pl.BlockSpec((2 * kh, f), lambda i: (0, 0)),
        ],
        out_specs=pl.BlockSpec((_BLK_M, f), lambda i: (i, 0)),
        out_shape=jax.ShapeDtypeStruct((n, f), jnp.float32),
    )(params, agg, w)


def _make_sc_agg(fh, e_pad):
    n = N_NODES
    edges_per_sub = e_pad // N_SUBCORES
    chunks_per_sub = edges_per_sub // CHUNK
    n_fill = 10
    rows_per_fill = n // n_fill
    btile = 40

    mesh = plsc.VectorSubcoreMesh(core_axis_name="c", subcore_axis_name="s")

    @functools.partial(
        pl.kernel,
        out_type=jax.ShapeDtypeStruct((2, n, fh), jnp.float32),
        mesh=mesh,
        compiler_params=pltpu.CompilerParams(needs_layout_passes=False),
        scratch_types=[
            pltpu.VMEM((CHUNK,), jnp.int32),
            pltpu.VMEM((CHUNK,), jnp.int32),
            pltpu.VMEM((CHUNK,), jnp.float32),
            pltpu.VMEM((CHUNK, fh), jnp.float32),
            pltpu.VMEM((btile, fh), jnp.float32),
            pltpu.VMEM_SHARED((n, fh), jnp.float32),
            pltpu.SemaphoreType.DMA,
        ],
    )
    def sc_agg(h_hbm, src_hbm, dst_hbm, w_hbm, bias_hbm, out_hbm,
               src_v, dst_v, w_v, rows_v, btile_v, acc_sh, sem):
        c = lax.axis_index("c")
        s = lax.axis_index("s")

        @pl.when(s < n_fill)
        def _init():
            pltpu.sync_copy(bias_hbm.at[c], btile_v.at[pl.ds(0, 1)])
            for j in range(fh // 16):
                sl = pl.ds(j * 16, 16)
                bv = btile_v[0, sl]
                for r in range(1, btile):
                    btile_v[r, sl] = bv
            for t in range(rows_per_fill // btile):
                pltpu.sync_copy(
                    btile_v, acc_sh.at[pl.ds(s * rows_per_fill + t * btile, btile)]
                )

        plsc.subcore_barrier()

        row_off = c * n
        base_s = s * edges_per_sub

        def chunk_body(k, carry):
            base = base_s + k * CHUNK
            pltpu.sync_copy(src_hbm.at[pl.ds(base, CHUNK)], src_v)
            pltpu.sync_copy(dst_hbm.at[pl.ds(base, CHUNK)], dst_v)
            pltpu.sync_copy(w_hbm.at[pl.ds(base, CHUNK)], w_v)
            for j in range(CHUNK // 16):
                sl = pl.ds(j * 16, 16)
                src_v[sl] = src_v[sl] + row_off
            pltpu.async_copy(h_hbm.at[src_v], rows_v, sem).wait()

            def edge_body(e, inner):
                ws = plsc.load_gather(w_v, [jnp.full((16,), e, jnp.int32)])
                for j in range(fh // 16):
                    sl = pl.ds(j * 16, 16)
                    rows_v[e, sl] = rows_v[e, sl] * ws
                return inner

            lax.fori_loop(0, CHUNK, edge_body, 0)

            pltpu.sync_copy(rows_v, acc_sh.at[dst_v], add=True)
            return carry

        lax.fori_loop(0, chunks_per_sub, chunk_body, 0)
        plsc.subcore_barrier()

        @pl.when(s < n_fill)
        def _drain():
            r0 = s * rows_per_fill
            pltpu.sync_copy(
                acc_sh.at[pl.ds(r0, rows_per_fill)],
                out_hbm.at[c].at[pl.ds(r0, rows_per_fill)],
            )

    return sc_agg


def _make_sc_agg_edgesplit(f, e_pad):
    n = N_NODES
    n_workers = 2 * N_SUBCORES
    edges_per_sub = e_pad // n_workers
    chunks_per_sub = edges_per_sub // CHUNK
    n_fill = 10
    rows_per_fill = n // n_fill
    btile = 40

    mesh = plsc.VectorSubcoreMesh(core_axis_name="c", subcore_axis_name="s")

    @functools.partial(
        pl.kernel,
        out_type=jax.ShapeDtypeStruct((2, n, f), jnp.float32),
        mesh=mesh,
        compiler_params=pltpu.CompilerParams(needs_layout_passes=False),
        scratch_types=[
            pltpu.VMEM((CHUNK,), jnp.int32),
            pltpu.VMEM((CHUNK,), jnp.int32),
            pltpu.VMEM((CHUNK,), jnp.float32),
            pltpu.VMEM((CHUNK, f), jnp.float32),
            pltpu.VMEM((btile, f), jnp.float32),
            pltpu.VMEM_SHARED((n, f), jnp.float32),
            pltpu.SemaphoreType.DMA,
        ],
    )
    def sc_agg(h_hbm, src_hbm, dst_hbm, w_hbm, bias_hbm, out_hbm,
               src_v, dst_v, w_v, rows_v, btile_v, acc_sh, sem):
        c = lax.axis_index("c")
        s = lax.axis_index("s")

        @pl.when(s < n_fill)
        def _init():
            pltpu.sync_copy(bias_hbm.at[c], btile_v.at[pl.ds(0, 1)])
            for j in range(f // 16):
                sl = pl.ds(j * 16, 16)
                bv = btile_v[0, sl]
                for r in range(1, btile):
                    btile_v[r, sl] = bv
            for t in range(rows_per_fill // btile):
                pltpu.sync_copy(
                    btile_v, acc_sh.at[pl.ds(s * rows_per_fill + t * btile, btile)]
                )

        plsc.subcore_barrier()

        base_s = (c * N_SUBCORES + s) * edges_per_sub

        def chunk_body(k, carry):
            base = base_s + k * CHUNK
            pltpu.sync_copy(src_hbm.at[pl.ds(base, CHUNK)], src_v)
            pltpu.sync_copy(dst_hbm.at[pl.ds(base, CHUNK)], dst_v)
            pltpu.sync_copy(w_hbm.at[pl.ds(base, CHUNK)], w_v)
            pltpu.async_copy(h_hbm.at[src_v], rows_v, sem).wait()

            def edge_body(e, inner):
                ws = plsc.load_gather(w_v, [jnp.full((16,), e, jnp.int32)])
                for j in range(f // 16):
                    sl = pl.ds(j * 16, 16)
                    rows_v[e, sl] = rows_v[e, sl] * ws
                return inner

            lax.fori_loop(0, CHUNK, edge_body, 0)
            pltpu.sync_copy(rows_v, acc_sh.at[dst_v], add=True)
            return carry

        lax.fori_loop(0, chunks_per_sub, chunk_body, 0)
        plsc.subcore_barrier()

        @pl.when(s < n_fill)
        def _drain():
            r0 = s * rows_per_fill
            pltpu.sync_copy(
                acc_sh.at[pl.ds(r0, rows_per_fill)],
                out_hbm.at[c].at[pl.ds(r0, rows_per_fill)],
            )

    return sc_agg


def _sum2_body(in_ref, o_ref):
    o_ref[...] = in_ref[0] + in_ref[1]


def _sum_partials(p):
    _, n, f = p.shape
    grid = (n // _BLK_M,)
    return pl.pallas_call(
        _sum2_body,
        grid=grid,
        in_specs=[pl.BlockSpec((2, _BLK_M, f), lambda i: (0, i, 0))],
        out_specs=pl.BlockSpec((_BLK_M, f), lambda i: (i, 0)),
        out_shape=jax.ShapeDtypeStruct((n, f), jnp.float32),
    )(p)


def kernel(x, edge_index, edge_weight, W1, b1, W2, b2, a, b, c, d):
    n = x.shape[0]
    e = edge_index.shape[1]
    quant = 2 * N_SUBCORES * CHUNK
    e_pad = ((e + quant - 1) // quant) * quant
    pad = e_pad - e

    src = jnp.concatenate([edge_index[0], jnp.zeros((pad,), jnp.int32)])
    dst = jnp.concatenate([edge_index[1], jnp.zeros((pad,), jnp.int32)])
    w = jnp.concatenate([edge_weight, jnp.zeros((pad,), jnp.float32)])

    h1 = _matmul_split(x, W1)
    agg1 = _make_sc_agg(HID // 2, e_pad)(
        h1.reshape(2 * n, HID // 2), src, dst, w, b1.reshape(2, 1, HID // 2)
    )

    params = jnp.stack([a, b, c, d])
    h2_full = _matmul2_full(params, agg1, W2)
    bias2 = jnp.stack([b2, jnp.zeros_like(b2)]).reshape(2, 1, D_OUT)
    parts = _make_sc_agg_edgesplit(D_OUT, e_pad)(
        h2_full, src, dst, w, bias2
    )
    return _sum_partials(parts)
